# Initial kernel scaffold; baseline (speedup 1.0000x reference)
import jax
import jax.numpy as jnp
from jax.experimental import pallas as pl


def kernel(x, edge_index, Win, b_in, ln_g, ln_b, W1, b1, W2, b2, out_g, out_b, Wout, b_out):
    raise NotImplementedError("write your pallas kernel here")


# speedup vs baseline: 4.2256x; 4.2256x over previous
import functools

import jax
import jax.numpy as jnp
from jax import lax
from jax.experimental import pallas as pl
from jax.experimental.pallas import tpu as pltpu
from jax.experimental.pallas import tpu_sc as plsc

N, E, D, H, L = 10000, 160000, 256, 512, 3
NC, NS = 2, 16
CH = 128
TCHUNKS = 1280
EPAD = TCHUNKS * CH
GARBAGE = N
ACC_ROWS = N + 16
ZPT = ACC_ROWS // NS
RPT = N // NS
HD = D // 2
BN = 1000
GRID = N // BN


@functools.cache
def _sc_kernels():
    mesh = plsc.VectorSubcoreMesh(
        core_axis_name="c", subcore_axis_name="s", num_cores=NC, num_subcores=NS
    )

    params = pltpu.CompilerParams(use_tc_tiling_on_sc=False)

    @functools.partial(
        pl.kernel,
        out_type=jax.ShapeDtypeStruct((NC * N, HD), jnp.float32),
        mesh=mesh,
        compiler_params=params,
        scratch_types=[
            pltpu.VMEM_SHARED((ACC_ROWS, HD), jnp.float32),
            pltpu.VMEM((CH,), jnp.int32),
            pltpu.VMEM((CH,), jnp.int32),
            pltpu.VMEM((CH, HD), jnp.float32),
            pltpu.SemaphoreType.DMA,
        ],
    )
    def _seg_sum(rp_hbm, sidx_hbm, didx_hbm, zeros_hbm, out_hbm,
                 acc, sidx_v, didx_v, rows_v, sem):
        c = lax.axis_index("c")
        s = lax.axis_index("s")
        pltpu.sync_copy(zeros_hbm.at[pl.ds(s * ZPT, ZPT)],
                        acc.at[pl.ds(s * ZPT, ZPT)])
        plsc.subcore_barrier()
        cpt = TCHUNKS // NS

        def body(j, carry):
            t = s * cpt + j
            pltpu.sync_copy(sidx_hbm.at[c * TCHUNKS + t], sidx_v)
            pltpu.sync_copy(didx_hbm.at[t], didx_v)
            pltpu.async_copy(rp_hbm.at[sidx_v], rows_v, sem).wait()
            pltpu.sync_copy(rows_v, acc.at[didx_v], add=True)
            return carry

        lax.fori_loop(0, cpt, body, 0)
        plsc.subcore_barrier()
        pltpu.sync_copy(acc.at[pl.ds(s * RPT, RPT)],
                        out_hbm.at[pl.ds(c * N + s * RPT, RPT)])

    @functools.partial(
        pl.kernel,
        out_type=jax.ShapeDtypeStruct((NC * N, 16), jnp.float32),
        mesh=mesh,
        compiler_params=params,
        scratch_types=[
            pltpu.VMEM_SHARED((ACC_ROWS, 16), jnp.float32),
            pltpu.VMEM((CH,), jnp.int32),
            pltpu.VMEM((CH, 16), jnp.float32),
        ],
    )
    def _deg_count(didx_hbm, zeros_hbm, ones_hbm, out_hbm, acc, idx_v, ones_v):
        c = lax.axis_index("c")
        s = lax.axis_index("s")
        pltpu.sync_copy(zeros_hbm.at[pl.ds(s * ZPT, ZPT)],
                        acc.at[pl.ds(s * ZPT, ZPT)])
        pltpu.sync_copy(ones_hbm, ones_v)
        plsc.subcore_barrier()
        cpt = TCHUNKS // (NC * NS)

        def body(j, carry):
            t = (c * NS + s) * cpt + j
            pltpu.sync_copy(didx_hbm.at[t], idx_v)
            pltpu.sync_copy(ones_v, acc.at[idx_v], add=True)
            return carry

        lax.fori_loop(0, cpt, body, 0)
        plsc.subcore_barrier()
        pltpu.sync_copy(acc.at[pl.ds(s * RPT, RPT)],
                        out_hbm.at[pl.ds(c * N + s * RPT, RPT)])

    return _seg_sum, _deg_count


_SQRT_HALF = 0.7071067811865476


def _gelu(x):
    return 0.5 * x * (1.0 + lax.erf(x * _SQRT_HALF))


def _ln(x, g, b):
    mu = jnp.mean(x, axis=-1, keepdims=True)
    var = jnp.mean((x - mu) ** 2, axis=-1, keepdims=True)
    return (x - mu) * lax.rsqrt(var + 1e-5) * g + b


def _dotT(a, w):
    return lax.dot_general(a, w, (((1,), (1,)), ((), ())),
                           preferred_element_type=jnp.float32)


def _dinv_from(deg_ref):
    deg = deg_ref[0, :, 0:1] + deg_ref[1, :, 0:1]
    return lax.rsqrt(jnp.maximum(deg, 1.0))


def _tc_in_body(x_ref, win_ref, bin_ref, deg_ref, g_ref, b_ref, h_ref, rp_ref):
    h = _gelu(_dotT(x_ref[...], win_ref[...]) + bin_ref[...])
    h_ref[...] = h
    r = _ln(h, g_ref[...], b_ref[...]) * _dinv_from(deg_ref)
    rp_ref[0] = r[:, :HD]
    rp_ref[1] = r[:, HD:]


def _tc_mid_body(h_ref, agg_ref, deg_ref, w1_ref, b1_ref, w2_ref, b2_ref,
                 g_ref, b_ref, hn_ref, rp_ref):
    dinv = _dinv_from(deg_ref)
    agg = jnp.concatenate([agg_ref[0], agg_ref[1]], axis=-1) * dinv
    f = _gelu(_dotT(agg, w1_ref[...]) + b1_ref[...])
    hn = h_ref[...] + _dotT(f, w2_ref[...]) + b2_ref[...]
    hn_ref[...] = hn
    r = _ln(hn, g_ref[...], b_ref[...]) * dinv
    rp_ref[0] = r[:, :HD]
    rp_ref[1] = r[:, HD:]


def _tc_out_body(h_ref, agg_ref, deg_ref, w1_ref, b1_ref, w2_ref, b2_ref,
                 g_ref, b_ref, wout_ref, bout_ref, o_ref):
    dinv = _dinv_from(deg_ref)
    agg = jnp.concatenate([agg_ref[0], agg_ref[1]], axis=-1) * dinv
    f = _gelu(_dotT(agg, w1_ref[...]) + b1_ref[...])
    hn = h_ref[...] + _dotT(f, w2_ref[...]) + b2_ref[...]
    r = _ln(hn, g_ref[...], b_ref[...])
    o_ref[...] = _dotT(r, wout_ref[...]) + bout_ref[...]


def _vec_spec(n):
    return pl.BlockSpec((n,), lambda i: (0,))


_ROW_SPEC = pl.BlockSpec((BN, D), lambda i: (i, 0))
_AGG_SPEC = pl.BlockSpec((2, BN, HD), lambda i: (0, i, 0))
_DEG_SPEC = pl.BlockSpec((2, BN, 16), lambda i: (0, i, 0))

_tc_in = pl.pallas_call(
    _tc_in_body,
    grid=(GRID,),
    in_specs=[
        _ROW_SPEC,
        pl.BlockSpec((D, D), lambda i: (0, 0)),
        _vec_spec(D),
        _DEG_SPEC,
        _vec_spec(D),
        _vec_spec(D),
    ],
    out_specs=[_ROW_SPEC, _AGG_SPEC],
    out_shape=[
        jax.ShapeDtypeStruct((N, D), jnp.float32),
        jax.ShapeDtypeStruct((2, N, HD), jnp.float32),
    ],
)

_tc_mid = pl.pallas_call(
    _tc_mid_body,
    grid=(GRID,),
    in_specs=[
        _ROW_SPEC,
        _AGG_SPEC,
        _DEG_SPEC,
        pl.BlockSpec((H, D), lambda i: (0, 0)),
        _vec_spec(H),
        pl.BlockSpec((D, H), lambda i: (0, 0)),
        _vec_spec(D),
        _vec_spec(D),
        _vec_spec(D),
    ],
    out_specs=[_ROW_SPEC, _AGG_SPEC],
    out_shape=[
        jax.ShapeDtypeStruct((N, D), jnp.float32),
        jax.ShapeDtypeStruct((2, N, HD), jnp.float32),
    ],
)

_tc_out = pl.pallas_call(
    _tc_out_body,
    grid=(GRID,),
    in_specs=[
        _ROW_SPEC,
        _AGG_SPEC,
        _DEG_SPEC,
        pl.BlockSpec((H, D), lambda i: (0, 0)),
        _vec_spec(H),
        pl.BlockSpec((D, H), lambda i: (0, 0)),
        _vec_spec(D),
        _vec_spec(D),
        _vec_spec(D),
        pl.BlockSpec((D, D), lambda i: (0, 0)),
        _vec_spec(D),
    ],
    out_specs=_ROW_SPEC,
    out_shape=jax.ShapeDtypeStruct((N, D), jnp.float32),
)


def kernel(x, edge_index, Win, b_in, ln_g, ln_b, W1, b1, W2, b2,
           out_g, out_b, Wout, b_out):
    seg_sum, deg_count = _sc_kernels()
    src = edge_index[0]
    dst = edge_index[1]
    pad = EPAD - E
    src0 = jnp.concatenate([src, jnp.zeros((pad,), jnp.int32)])
    sidx = jnp.concatenate([src0, src0 + N]).reshape(NC * TCHUNKS, CH)
    didx = jnp.concatenate(
        [dst, jnp.full((pad,), GARBAGE, jnp.int32)]).reshape(TCHUNKS, CH)
    degidx = jnp.concatenate(
        [src, jnp.full((pad,), GARBAGE, jnp.int32)]).reshape(TCHUNKS, CH)
    zeros_hd = jnp.zeros((ACC_ROWS, HD), jnp.float32)
    zeros_16 = jnp.zeros((ACC_ROWS, 16), jnp.float32)
    ones_16 = jnp.ones((CH, 16), jnp.float32)

    deg2 = deg_count(degidx, zeros_16, ones_16).reshape(NC, N, 16)
    h, rp = _tc_in(x, Win, b_in, deg2, ln_g[0], ln_b[0])
    out = None
    for l in range(L):
        agg2 = seg_sum(rp.reshape(NC * N, HD), sidx, didx,
                       zeros_hd).reshape(NC, N, HD)
        if l < L - 1:
            h, rp = _tc_mid(h, agg2, deg2, W1[l], b1[l], W2[l], b2[l],
                            ln_g[l + 1], ln_b[l + 1])
        else:
            out = _tc_out(h, agg2, deg2, W1[l], b1[l], W2[l], b2[l],
                          out_g, out_b, Wout, b_out)
    return out


# speedup vs baseline: 5.8063x; 1.3741x over previous
import functools

import jax
import jax.numpy as jnp
from jax import lax
from jax.experimental import pallas as pl
from jax.experimental.pallas import tpu as pltpu
from jax.experimental.pallas import tpu_sc as plsc

N, E, D, H, L = 10000, 160000, 256, 512, 3
NC, NS = 2, 16
CH = 128
TCHUNKS = 1280
EPAD = TCHUNKS * CH
GARBAGE = N
ACC_ROWS = N + 16
ZPT = ACC_ROWS // NS
RPT = N // NS
HD = D // 2
BN = 1000
GRID = N // BN


@functools.cache
def _sc_kernels():
    mesh = plsc.VectorSubcoreMesh(
        core_axis_name="c", subcore_axis_name="s", num_cores=NC, num_subcores=NS
    )

    params = pltpu.CompilerParams(use_tc_tiling_on_sc=False)

    cpt = TCHUNKS // NS
    half = cpt // 2
    nbuf = 2

    @functools.partial(
        pl.kernel,
        out_type=jax.ShapeDtypeStruct((NC * N, HD), jnp.float32),
        mesh=mesh,
        compiler_params=params,
        scratch_types=[
            pltpu.VMEM_SHARED((ACC_ROWS, HD), jnp.float32),
            pltpu.VMEM((half, CH), jnp.int32),
            pltpu.VMEM((half, CH), jnp.int32),
            [pltpu.VMEM((CH, HD), jnp.float32)] * nbuf,
            [pltpu.SemaphoreType.DMA] * nbuf,
        ],
    )
    def _seg_sum(rp_hbm, sidx_hbm, didx_hbm, zeros_hbm, out_hbm,
                 acc, sidx_buf, didx_buf, rows, sems):
        c = lax.axis_index("c")
        s = lax.axis_index("s")
        pltpu.sync_copy(zeros_hbm.at[pl.ds(s * ZPT, ZPT)],
                        acc.at[pl.ds(s * ZPT, ZPT)])
        plsc.subcore_barrier()
        for hf in range(2):
            pltpu.sync_copy(
                sidx_hbm.at[pl.ds(c * TCHUNKS + s * cpt + hf * half, half)],
                sidx_buf)
            pltpu.sync_copy(didx_hbm.at[pl.ds(s * cpt + hf * half, half)],
                            didx_buf)
            for b in range(nbuf):
                pltpu.async_copy(rp_hbm.at[sidx_buf.at[b]], rows[b], sems[b])

            @pl.loop(0, half, step=nbuf)
            def _group(g):
                for b in range(nbuf):
                    j = g + b
                    pltpu.make_async_copy(rp_hbm.at[sidx_buf.at[b]],
                                          rows[b], sems[b]).wait()
                    pltpu.sync_copy(rows[b], acc.at[didx_buf.at[j]], add=True)

                    @pl.when(j + nbuf < half)
                    def _prefetch():
                        pltpu.async_copy(rp_hbm.at[sidx_buf.at[j + nbuf]],
                                         rows[b], sems[b])

        plsc.subcore_barrier()
        pltpu.sync_copy(acc.at[pl.ds(s * RPT, RPT)],
                        out_hbm.at[pl.ds(c * N + s * RPT, RPT)])

    @functools.partial(
        pl.kernel,
        out_type=jax.ShapeDtypeStruct((NC * N, 16), jnp.float32),
        mesh=mesh,
        compiler_params=params,
        scratch_types=[
            pltpu.VMEM_SHARED((ACC_ROWS, 16), jnp.float32),
            pltpu.VMEM((CH,), jnp.int32),
            pltpu.VMEM((CH, 16), jnp.float32),
        ],
    )
    def _deg_count(didx_hbm, zeros_hbm, ones_hbm, out_hbm, acc, idx_v, ones_v):
        c = lax.axis_index("c")
        s = lax.axis_index("s")
        pltpu.sync_copy(zeros_hbm.at[pl.ds(s * ZPT, ZPT)],
                        acc.at[pl.ds(s * ZPT, ZPT)])
        pltpu.sync_copy(ones_hbm, ones_v)
        plsc.subcore_barrier()
        cpt = TCHUNKS // (NC * NS)

        def body(j, carry):
            t = (c * NS + s) * cpt + j
            pltpu.sync_copy(didx_hbm.at[t], idx_v)
            pltpu.sync_copy(ones_v, acc.at[idx_v], add=True)
            return carry

        lax.fori_loop(0, cpt, body, 0)
        plsc.subcore_barrier()
        pltpu.sync_copy(acc.at[pl.ds(s * RPT, RPT)],
                        out_hbm.at[pl.ds(c * N + s * RPT, RPT)])

    return _seg_sum, _deg_count


_SQRT_HALF = 0.7071067811865476


def _gelu(x):
    return 0.5 * x * (1.0 + lax.erf(x * _SQRT_HALF))


def _ln(x, g, b):
    mu = jnp.mean(x, axis=-1, keepdims=True)
    var = jnp.mean((x - mu) ** 2, axis=-1, keepdims=True)
    return (x - mu) * lax.rsqrt(var + 1e-5) * g + b


def _dotT(a, w):
    return lax.dot_general(a, w, (((1,), (1,)), ((), ())),
                           preferred_element_type=jnp.float32)


def _dinv_from(deg_ref):
    deg = deg_ref[0, :, 0:1] + deg_ref[1, :, 0:1]
    return lax.rsqrt(jnp.maximum(deg, 1.0))


def _tc_in_body(x_ref, win_ref, bin_ref, deg_ref, g_ref, b_ref, h_ref, rp_ref):
    h = _gelu(_dotT(x_ref[...], win_ref[...]) + bin_ref[...])
    h_ref[...] = h
    r = _ln(h, g_ref[...], b_ref[...]) * _dinv_from(deg_ref)
    rp_ref[0] = r[:, :HD]
    rp_ref[1] = r[:, HD:]


def _tc_mid_body(h_ref, agg_ref, deg_ref, w1_ref, b1_ref, w2_ref, b2_ref,
                 g_ref, b_ref, hn_ref, rp_ref):
    dinv = _dinv_from(deg_ref)
    agg = jnp.concatenate([agg_ref[0], agg_ref[1]], axis=-1) * dinv
    f = _gelu(_dotT(agg, w1_ref[...]) + b1_ref[...])
    hn = h_ref[...] + _dotT(f, w2_ref[...]) + b2_ref[...]
    hn_ref[...] = hn
    r = _ln(hn, g_ref[...], b_ref[...]) * dinv
    rp_ref[0] = r[:, :HD]
    rp_ref[1] = r[:, HD:]


def _tc_out_body(h_ref, agg_ref, deg_ref, w1_ref, b1_ref, w2_ref, b2_ref,
                 g_ref, b_ref, wout_ref, bout_ref, o_ref):
    dinv = _dinv_from(deg_ref)
    agg = jnp.concatenate([agg_ref[0], agg_ref[1]], axis=-1) * dinv
    f = _gelu(_dotT(agg, w1_ref[...]) + b1_ref[...])
    hn = h_ref[...] + _dotT(f, w2_ref[...]) + b2_ref[...]
    r = _ln(hn, g_ref[...], b_ref[...])
    o_ref[...] = _dotT(r, wout_ref[...]) + bout_ref[...]


def _vec_spec(n):
    return pl.BlockSpec((n,), lambda i: (0,))


_ROW_SPEC = pl.BlockSpec((BN, D), lambda i: (i, 0))
_AGG_SPEC = pl.BlockSpec((2, BN, HD), lambda i: (0, i, 0))
_DEG_SPEC = pl.BlockSpec((2, BN, 16), lambda i: (0, i, 0))

_tc_in = pl.pallas_call(
    _tc_in_body,
    grid=(GRID,),
    in_specs=[
        _ROW_SPEC,
        pl.BlockSpec((D, D), lambda i: (0, 0)),
        _vec_spec(D),
        _DEG_SPEC,
        _vec_spec(D),
        _vec_spec(D),
    ],
    out_specs=[_ROW_SPEC, _AGG_SPEC],
    out_shape=[
        jax.ShapeDtypeStruct((N, D), jnp.float32),
        jax.ShapeDtypeStruct((2, N, HD), jnp.float32),
    ],
)

_tc_mid = pl.pallas_call(
    _tc_mid_body,
    grid=(GRID,),
    in_specs=[
        _ROW_SPEC,
        _AGG_SPEC,
        _DEG_SPEC,
        pl.BlockSpec((H, D), lambda i: (0, 0)),
        _vec_spec(H),
        pl.BlockSpec((D, H), lambda i: (0, 0)),
        _vec_spec(D),
        _vec_spec(D),
        _vec_spec(D),
    ],
    out_specs=[_ROW_SPEC, _AGG_SPEC],
    out_shape=[
        jax.ShapeDtypeStruct((N, D), jnp.float32),
        jax.ShapeDtypeStruct((2, N, HD), jnp.float32),
    ],
)

_tc_out = pl.pallas_call(
    _tc_out_body,
    grid=(GRID,),
    in_specs=[
        _ROW_SPEC,
        _AGG_SPEC,
        _DEG_SPEC,
        pl.BlockSpec((H, D), lambda i: (0, 0)),
        _vec_spec(H),
        pl.BlockSpec((D, H), lambda i: (0, 0)),
        _vec_spec(D),
        _vec_spec(D),
        _vec_spec(D),
        pl.BlockSpec((D, D), lambda i: (0, 0)),
        _vec_spec(D),
    ],
    out_specs=_ROW_SPEC,
    out_shape=jax.ShapeDtypeStruct((N, D), jnp.float32),
)


def kernel(x, edge_index, Win, b_in, ln_g, ln_b, W1, b1, W2, b2,
           out_g, out_b, Wout, b_out):
    seg_sum, deg_count = _sc_kernels()
    src = edge_index[0]
    dst = edge_index[1]
    pad = EPAD - E
    src0 = jnp.concatenate([src, jnp.zeros((pad,), jnp.int32)])
    sidx = jnp.concatenate([src0, src0 + N]).reshape(NC * TCHUNKS, CH)
    didx = jnp.concatenate(
        [dst, jnp.full((pad,), GARBAGE, jnp.int32)]).reshape(TCHUNKS, CH)
    degidx = jnp.concatenate(
        [src, jnp.full((pad,), GARBAGE, jnp.int32)]).reshape(TCHUNKS, CH)
    zeros_hd = jnp.zeros((ACC_ROWS, HD), jnp.float32)
    zeros_16 = jnp.zeros((ACC_ROWS, 16), jnp.float32)
    ones_16 = jnp.ones((CH, 16), jnp.float32)

    deg2 = deg_count(degidx, zeros_16, ones_16).reshape(NC, N, 16)
    h, rp = _tc_in(x, Win, b_in, deg2, ln_g[0], ln_b[0])
    out = None
    for l in range(L):
        agg2 = seg_sum(rp.reshape(NC * N, HD), sidx, didx,
                       zeros_hd).reshape(NC, N, HD)
        if l < L - 1:
            h, rp = _tc_mid(h, agg2, deg2, W1[l], b1[l], W2[l], b2[l],
                            ln_g[l + 1], ln_b[l + 1])
        else:
            out = _tc_out(h, agg2, deg2, W1[l], b1[l], W2[l], b2[l],
                          out_g, out_b, Wout, b_out)
    return out


# speedup vs baseline: 5.8160x; 1.0017x over previous
import functools

import jax
import jax.numpy as jnp
from jax import lax
from jax.experimental import pallas as pl
from jax.experimental.pallas import tpu as pltpu
from jax.experimental.pallas import tpu_sc as plsc

N, E, D, H, L = 10000, 160000, 256, 512, 3
NC, NS = 2, 16
CH = 128
TCHUNKS = 1280
EPAD = TCHUNKS * CH
GARBAGE = N
ACC_ROWS = N + 16
ZPT = ACC_ROWS // NS
RPT = N // NS
HD = D // 2
BN = 1000
GRID = N // BN


@functools.cache
def _sc_kernels():
    mesh = plsc.VectorSubcoreMesh(
        core_axis_name="c", subcore_axis_name="s", num_cores=NC, num_subcores=NS
    )

    params = pltpu.CompilerParams(use_tc_tiling_on_sc=False)

    cpt = TCHUNKS // NS
    half = cpt // 2
    nbuf = 2

    @functools.partial(
        pl.kernel,
        out_type=jax.ShapeDtypeStruct((NC * N, HD), jnp.float32),
        mesh=mesh,
        compiler_params=params,
        scratch_types=[
            pltpu.VMEM_SHARED((ACC_ROWS, HD), jnp.float32),
            pltpu.VMEM((half, CH), jnp.int32),
            pltpu.VMEM((half, CH), jnp.int32),
            [pltpu.VMEM((CH, HD), jnp.float32)] * nbuf,
            [pltpu.SemaphoreType.DMA] * nbuf,
            [pltpu.SemaphoreType.DMA] * nbuf,
        ],
    )
    def _seg_sum(rp_hbm, sidx_hbm, didx_hbm, zeros_hbm, out_hbm,
                 acc, sidx_buf, didx_buf, rows, gsems, ssems):
        c = lax.axis_index("c")
        s = lax.axis_index("s")

        def gather(j, b):
            pltpu.async_copy(rp_hbm.at[sidx_buf.at[j]], rows[b], gsems[b])

        def gather_wait(b):
            pltpu.make_async_copy(rp_hbm.at[sidx_buf.at[0]], rows[b],
                                  gsems[b]).wait()

        def scatter(j, b):
            pltpu.async_copy(rows[b], acc.at[didx_buf.at[j]], ssems[b],
                             add=True)

        def scatter_wait(b):
            pltpu.make_async_copy(rows[b], acc.at[didx_buf.at[0]],
                                  ssems[b]).wait()

        pltpu.sync_copy(zeros_hbm.at[pl.ds(s * ZPT, ZPT)],
                        acc.at[pl.ds(s * ZPT, ZPT)])
        plsc.subcore_barrier()
        for hf in range(2):
            pltpu.sync_copy(
                sidx_hbm.at[pl.ds(c * TCHUNKS + s * cpt + hf * half, half)],
                sidx_buf)
            pltpu.sync_copy(didx_hbm.at[pl.ds(s * cpt + hf * half, half)],
                            didx_buf)
            gather(0, 0)

            @pl.loop(0, half, step=2)
            def _pair(g):
                @pl.when(g > 0)
                def _():
                    scatter_wait(1)

                gather(g + 1, 1)
                gather_wait(0)
                scatter(g, 0)

                @pl.when(g + 2 < half)
                def _():
                    scatter_wait(0)
                    gather(g + 2, 0)

                gather_wait(1)
                scatter(g + 1, 1)

            scatter_wait(0)
            scatter_wait(1)

        plsc.subcore_barrier()
        pltpu.sync_copy(acc.at[pl.ds(s * RPT, RPT)],
                        out_hbm.at[pl.ds(c * N + s * RPT, RPT)])

    @functools.partial(
        pl.kernel,
        out_type=jax.ShapeDtypeStruct((NC * N, 16), jnp.float32),
        mesh=mesh,
        compiler_params=params,
        scratch_types=[
            pltpu.VMEM_SHARED((ACC_ROWS, 16), jnp.float32),
            pltpu.VMEM((CH,), jnp.int32),
            pltpu.VMEM((CH, 16), jnp.float32),
        ],
    )
    def _deg_count(didx_hbm, zeros_hbm, ones_hbm, out_hbm, acc, idx_v, ones_v):
        c = lax.axis_index("c")
        s = lax.axis_index("s")
        pltpu.sync_copy(zeros_hbm.at[pl.ds(s * ZPT, ZPT)],
                        acc.at[pl.ds(s * ZPT, ZPT)])
        pltpu.sync_copy(ones_hbm, ones_v)
        plsc.subcore_barrier()
        cpt = TCHUNKS // (NC * NS)

        def body(j, carry):
            t = (c * NS + s) * cpt + j
            pltpu.sync_copy(didx_hbm.at[t], idx_v)
            pltpu.sync_copy(ones_v, acc.at[idx_v], add=True)
            return carry

        lax.fori_loop(0, cpt, body, 0)
        plsc.subcore_barrier()
        pltpu.sync_copy(acc.at[pl.ds(s * RPT, RPT)],
                        out_hbm.at[pl.ds(c * N + s * RPT, RPT)])

    return _seg_sum, _deg_count


_SQRT_HALF = 0.7071067811865476


def _gelu(x):
    return 0.5 * x * (1.0 + lax.erf(x * _SQRT_HALF))


def _ln(x, g, b):
    mu = jnp.mean(x, axis=-1, keepdims=True)
    var = jnp.mean((x - mu) ** 2, axis=-1, keepdims=True)
    return (x - mu) * lax.rsqrt(var + 1e-5) * g + b


def _dotT(a, w):
    return lax.dot_general(a, w, (((1,), (1,)), ((), ())),
                           preferred_element_type=jnp.float32)


def _dinv_from(deg_ref):
    deg = deg_ref[0, :, 0:1] + deg_ref[1, :, 0:1]
    return lax.rsqrt(jnp.maximum(deg, 1.0))


def _tc_in_body(x_ref, win_ref, bin_ref, deg_ref, g_ref, b_ref, h_ref, rp_ref):
    h = _gelu(_dotT(x_ref[...], win_ref[...]) + bin_ref[...])
    h_ref[...] = h
    r = _ln(h, g_ref[...], b_ref[...]) * _dinv_from(deg_ref)
    rp_ref[0] = r[:, :HD]
    rp_ref[1] = r[:, HD:]


def _tc_mid_body(h_ref, agg_ref, deg_ref, w1_ref, b1_ref, w2_ref, b2_ref,
                 g_ref, b_ref, hn_ref, rp_ref):
    dinv = _dinv_from(deg_ref)
    agg = jnp.concatenate([agg_ref[0], agg_ref[1]], axis=-1) * dinv
    f = _gelu(_dotT(agg, w1_ref[...]) + b1_ref[...])
    hn = h_ref[...] + _dotT(f, w2_ref[...]) + b2_ref[...]
    hn_ref[...] = hn
    r = _ln(hn, g_ref[...], b_ref[...]) * dinv
    rp_ref[0] = r[:, :HD]
    rp_ref[1] = r[:, HD:]


def _tc_out_body(h_ref, agg_ref, deg_ref, w1_ref, b1_ref, w2_ref, b2_ref,
                 g_ref, b_ref, wout_ref, bout_ref, o_ref):
    dinv = _dinv_from(deg_ref)
    agg = jnp.concatenate([agg_ref[0], agg_ref[1]], axis=-1) * dinv
    f = _gelu(_dotT(agg, w1_ref[...]) + b1_ref[...])
    hn = h_ref[...] + _dotT(f, w2_ref[...]) + b2_ref[...]
    r = _ln(hn, g_ref[...], b_ref[...])
    o_ref[...] = _dotT(r, wout_ref[...]) + bout_ref[...]


def _vec_spec(n):
    return pl.BlockSpec((n,), lambda i: (0,))


_ROW_SPEC = pl.BlockSpec((BN, D), lambda i: (i, 0))
_AGG_SPEC = pl.BlockSpec((2, BN, HD), lambda i: (0, i, 0))
_DEG_SPEC = pl.BlockSpec((2, BN, 16), lambda i: (0, i, 0))

_tc_in = pl.pallas_call(
    _tc_in_body,
    grid=(GRID,),
    in_specs=[
        _ROW_SPEC,
        pl.BlockSpec((D, D), lambda i: (0, 0)),
        _vec_spec(D),
        _DEG_SPEC,
        _vec_spec(D),
        _vec_spec(D),
    ],
    out_specs=[_ROW_SPEC, _AGG_SPEC],
    out_shape=[
        jax.ShapeDtypeStruct((N, D), jnp.float32),
        jax.ShapeDtypeStruct((2, N, HD), jnp.float32),
    ],
)

_tc_mid = pl.pallas_call(
    _tc_mid_body,
    grid=(GRID,),
    in_specs=[
        _ROW_SPEC,
        _AGG_SPEC,
        _DEG_SPEC,
        pl.BlockSpec((H, D), lambda i: (0, 0)),
        _vec_spec(H),
        pl.BlockSpec((D, H), lambda i: (0, 0)),
        _vec_spec(D),
        _vec_spec(D),
        _vec_spec(D),
    ],
    out_specs=[_ROW_SPEC, _AGG_SPEC],
    out_shape=[
        jax.ShapeDtypeStruct((N, D), jnp.float32),
        jax.ShapeDtypeStruct((2, N, HD), jnp.float32),
    ],
)

_tc_out = pl.pallas_call(
    _tc_out_body,
    grid=(GRID,),
    in_specs=[
        _ROW_SPEC,
        _AGG_SPEC,
        _DEG_SPEC,
        pl.BlockSpec((H, D), lambda i: (0, 0)),
        _vec_spec(H),
        pl.BlockSpec((D, H), lambda i: (0, 0)),
        _vec_spec(D),
        _vec_spec(D),
        _vec_spec(D),
        pl.BlockSpec((D, D), lambda i: (0, 0)),
        _vec_spec(D),
    ],
    out_specs=_ROW_SPEC,
    out_shape=jax.ShapeDtypeStruct((N, D), jnp.float32),
)


def kernel(x, edge_index, Win, b_in, ln_g, ln_b, W1, b1, W2, b2,
           out_g, out_b, Wout, b_out):
    seg_sum, deg_count = _sc_kernels()
    src = edge_index[0]
    dst = edge_index[1]
    pad = EPAD - E
    src0 = jnp.concatenate([src, jnp.zeros((pad,), jnp.int32)])
    sidx = jnp.concatenate([src0, src0 + N]).reshape(NC * TCHUNKS, CH)
    didx = jnp.concatenate(
        [dst, jnp.full((pad,), GARBAGE, jnp.int32)]).reshape(TCHUNKS, CH)
    degidx = jnp.concatenate(
        [src, jnp.full((pad,), GARBAGE, jnp.int32)]).reshape(TCHUNKS, CH)
    zeros_hd = jnp.zeros((ACC_ROWS, HD), jnp.float32)
    zeros_16 = jnp.zeros((ACC_ROWS, 16), jnp.float32)
    ones_16 = jnp.ones((CH, 16), jnp.float32)

    deg2 = deg_count(degidx, zeros_16, ones_16).reshape(NC, N, 16)
    h, rp = _tc_in(x, Win, b_in, deg2, ln_g[0], ln_b[0])
    out = None
    for l in range(L):
        agg2 = seg_sum(rp.reshape(NC * N, HD), sidx, didx,
                       zeros_hd).reshape(NC, N, HD)
        if l < L - 1:
            h, rp = _tc_mid(h, agg2, deg2, W1[l], b1[l], W2[l], b2[l],
                            ln_g[l + 1], ln_b[l + 1])
        else:
            out = _tc_out(h, agg2, deg2, W1[l], b1[l], W2[l], b2[l],
                          out_g, out_b, Wout, b_out)
    return out


# speedup vs baseline: 8.0440x; 1.3831x over previous
import functools

import jax
import jax.numpy as jnp
from jax import lax
from jax.experimental import pallas as pl
from jax.experimental.pallas import tpu as pltpu
from jax.experimental.pallas import tpu_sc as plsc

N, E, D, H, L = 10000, 160000, 256, 512, 3
NC, NS = 2, 16
CH = 128
TCHUNKS = 1280
EPAD = TCHUNKS * CH
GARBAGE = N
ACC_ROWS = N + 16
ZPT = ACC_ROWS // NS
RPT = N // NS
HD = D // 2
BN = 1000
GRID = N // BN


@functools.cache
def _sc_kernels():
    mesh = plsc.VectorSubcoreMesh(
        core_axis_name="c", subcore_axis_name="s", num_cores=NC, num_subcores=NS
    )

    params = pltpu.CompilerParams(use_tc_tiling_on_sc=False)

    cpt = TCHUNKS // NS
    nbuf = 2

    @functools.partial(
        pl.kernel,
        out_type=jax.ShapeDtypeStruct((NC * N, HD), jnp.bfloat16),
        mesh=mesh,
        compiler_params=params,
        scratch_types=[
            pltpu.VMEM_SHARED((ACC_ROWS, HD), jnp.bfloat16),
            pltpu.VMEM((cpt, CH), jnp.int32),
            pltpu.VMEM((cpt, CH), jnp.int32),
            [pltpu.VMEM((CH, HD), jnp.bfloat16)] * nbuf,
            [pltpu.SemaphoreType.DMA] * nbuf,
            [pltpu.SemaphoreType.DMA] * nbuf,
        ],
    )
    def _seg_sum(rp_hbm, sidx_hbm, didx_hbm, zeros_hbm, out_hbm,
                 acc, sidx_buf, didx_buf, rows, gsems, ssems):
        c = lax.axis_index("c")
        s = lax.axis_index("s")

        def gather(j, b):
            pltpu.async_copy(rp_hbm.at[sidx_buf.at[j]], rows[b], gsems[b])

        def gather_wait(b):
            pltpu.make_async_copy(rp_hbm.at[sidx_buf.at[0]], rows[b],
                                  gsems[b]).wait()

        def scatter(j, b):
            pltpu.async_copy(rows[b], acc.at[didx_buf.at[j]], ssems[b],
                             add=True)

        def scatter_wait(b):
            pltpu.make_async_copy(rows[b], acc.at[didx_buf.at[0]],
                                  ssems[b]).wait()

        pltpu.sync_copy(zeros_hbm.at[pl.ds(s * ZPT, ZPT)],
                        acc.at[pl.ds(s * ZPT, ZPT)])
        pltpu.sync_copy(sidx_hbm.at[pl.ds(c * TCHUNKS + s * cpt, cpt)],
                        sidx_buf)
        pltpu.sync_copy(didx_hbm.at[pl.ds(s * cpt, cpt)], didx_buf)
        plsc.subcore_barrier()
        gather(0, 0)

        @pl.loop(0, cpt, step=2)
        def _pair(g):
            @pl.when(g > 0)
            def _():
                scatter_wait(1)

            gather(g + 1, 1)
            gather_wait(0)
            scatter(g, 0)

            @pl.when(g + 2 < cpt)
            def _():
                scatter_wait(0)
                gather(g + 2, 0)

            gather_wait(1)
            scatter(g + 1, 1)

        scatter_wait(0)
        scatter_wait(1)
        plsc.subcore_barrier()
        pltpu.sync_copy(acc.at[pl.ds(s * RPT, RPT)],
                        out_hbm.at[pl.ds(c * N + s * RPT, RPT)])

    @functools.partial(
        pl.kernel,
        out_type=jax.ShapeDtypeStruct((NC * N, 16), jnp.float32),
        mesh=mesh,
        compiler_params=params,
        scratch_types=[
            pltpu.VMEM_SHARED((ACC_ROWS, 16), jnp.float32),
            pltpu.VMEM((CH,), jnp.int32),
            pltpu.VMEM((CH, 16), jnp.float32),
        ],
    )
    def _deg_count(didx_hbm, zeros_hbm, ones_hbm, out_hbm, acc, idx_v, ones_v):
        c = lax.axis_index("c")
        s = lax.axis_index("s")
        pltpu.sync_copy(zeros_hbm.at[pl.ds(s * ZPT, ZPT)],
                        acc.at[pl.ds(s * ZPT, ZPT)])
        pltpu.sync_copy(ones_hbm, ones_v)
        plsc.subcore_barrier()
        cpt = TCHUNKS // (NC * NS)

        def body(j, carry):
            t = (c * NS + s) * cpt + j
            pltpu.sync_copy(didx_hbm.at[t], idx_v)
            pltpu.sync_copy(ones_v, acc.at[idx_v], add=True)
            return carry

        lax.fori_loop(0, cpt, body, 0)
        plsc.subcore_barrier()
        pltpu.sync_copy(acc.at[pl.ds(s * RPT, RPT)],
                        out_hbm.at[pl.ds(c * N + s * RPT, RPT)])

    return _seg_sum, _deg_count


_SQRT_HALF = 0.7071067811865476


def _gelu(x):
    return 0.5 * x * (1.0 + lax.erf(x * _SQRT_HALF))


def _ln(x, g, b):
    mu = jnp.mean(x, axis=-1, keepdims=True)
    var = jnp.mean((x - mu) ** 2, axis=-1, keepdims=True)
    return (x - mu) * lax.rsqrt(var + 1e-5) * g + b


def _dotT(a, w):
    return lax.dot_general(a, w, (((1,), (1,)), ((), ())),
                           preferred_element_type=jnp.float32)


def _dinv_from(deg_ref):
    deg = deg_ref[0, :, 0:1] + deg_ref[1, :, 0:1]
    return lax.rsqrt(jnp.maximum(deg, 1.0))


def _tc_in_body(x_ref, win_ref, bin_ref, deg_ref, g_ref, b_ref, h_ref, rp_ref):
    h = _gelu(_dotT(x_ref[...], win_ref[...]) + bin_ref[...])
    h_ref[...] = h
    r = (_ln(h, g_ref[...], b_ref[...]) * _dinv_from(deg_ref)
         ).astype(jnp.bfloat16)
    rp_ref[0] = r[:, :HD]
    rp_ref[1] = r[:, HD:]


def _tc_mid_body(h_ref, agg_ref, deg_ref, w1_ref, b1_ref, w2_ref, b2_ref,
                 g_ref, b_ref, hn_ref, rp_ref):
    dinv = _dinv_from(deg_ref)
    agg = jnp.concatenate([agg_ref[0], agg_ref[1]],
                          axis=-1).astype(jnp.float32) * dinv
    f = _gelu(_dotT(agg, w1_ref[...]) + b1_ref[...])
    hn = h_ref[...] + _dotT(f, w2_ref[...]) + b2_ref[...]
    hn_ref[...] = hn
    r = (_ln(hn, g_ref[...], b_ref[...]) * dinv).astype(jnp.bfloat16)
    rp_ref[0] = r[:, :HD]
    rp_ref[1] = r[:, HD:]


def _tc_out_body(h_ref, agg_ref, deg_ref, w1_ref, b1_ref, w2_ref, b2_ref,
                 g_ref, b_ref, wout_ref, bout_ref, o_ref):
    dinv = _dinv_from(deg_ref)
    agg = jnp.concatenate([agg_ref[0], agg_ref[1]],
                          axis=-1).astype(jnp.float32) * dinv
    f = _gelu(_dotT(agg, w1_ref[...]) + b1_ref[...])
    hn = h_ref[...] + _dotT(f, w2_ref[...]) + b2_ref[...]
    r = _ln(hn, g_ref[...], b_ref[...])
    o_ref[...] = _dotT(r, wout_ref[...]) + bout_ref[...]


def _vec_spec(n):
    return pl.BlockSpec((n,), lambda i: (0,))


_ROW_SPEC = pl.BlockSpec((BN, D), lambda i: (i, 0))
_AGG_SPEC = pl.BlockSpec((2, BN, HD), lambda i: (0, i, 0))
_DEG_SPEC = pl.BlockSpec((2, BN, 16), lambda i: (0, i, 0))

_tc_in = pl.pallas_call(
    _tc_in_body,
    grid=(GRID,),
    in_specs=[
        _ROW_SPEC,
        pl.BlockSpec((D, D), lambda i: (0, 0)),
        _vec_spec(D),
        _DEG_SPEC,
        _vec_spec(D),
        _vec_spec(D),
    ],
    out_specs=[_ROW_SPEC, _AGG_SPEC],
    out_shape=[
        jax.ShapeDtypeStruct((N, D), jnp.float32),
        jax.ShapeDtypeStruct((2, N, HD), jnp.bfloat16),
    ],
)

_tc_mid = pl.pallas_call(
    _tc_mid_body,
    grid=(GRID,),
    in_specs=[
        _ROW_SPEC,
        _AGG_SPEC,
        _DEG_SPEC,
        pl.BlockSpec((H, D), lambda i: (0, 0)),
        _vec_spec(H),
        pl.BlockSpec((D, H), lambda i: (0, 0)),
        _vec_spec(D),
        _vec_spec(D),
        _vec_spec(D),
    ],
    out_specs=[_ROW_SPEC, _AGG_SPEC],
    out_shape=[
        jax.ShapeDtypeStruct((N, D), jnp.float32),
        jax.ShapeDtypeStruct((2, N, HD), jnp.bfloat16),
    ],
)

_tc_out = pl.pallas_call(
    _tc_out_body,
    grid=(GRID,),
    in_specs=[
        _ROW_SPEC,
        _AGG_SPEC,
        _DEG_SPEC,
        pl.BlockSpec((H, D), lambda i: (0, 0)),
        _vec_spec(H),
        pl.BlockSpec((D, H), lambda i: (0, 0)),
        _vec_spec(D),
        _vec_spec(D),
        _vec_spec(D),
        pl.BlockSpec((D, D), lambda i: (0, 0)),
        _vec_spec(D),
    ],
    out_specs=_ROW_SPEC,
    out_shape=jax.ShapeDtypeStruct((N, D), jnp.float32),
)


def kernel(x, edge_index, Win, b_in, ln_g, ln_b, W1, b1, W2, b2,
           out_g, out_b, Wout, b_out):
    seg_sum, deg_count = _sc_kernels()
    src = edge_index[0]
    dst = edge_index[1]
    pad = EPAD - E
    src0 = jnp.concatenate([src, jnp.zeros((pad,), jnp.int32)])
    sidx = jnp.concatenate([src0, src0 + N]).reshape(NC * TCHUNKS, CH)
    didx = jnp.concatenate(
        [dst, jnp.full((pad,), GARBAGE, jnp.int32)]).reshape(TCHUNKS, CH)
    degidx = jnp.concatenate(
        [src, jnp.full((pad,), GARBAGE, jnp.int32)]).reshape(TCHUNKS, CH)
    zeros_hd = jnp.zeros((ACC_ROWS, HD), jnp.bfloat16)
    zeros_16 = jnp.zeros((ACC_ROWS, 16), jnp.float32)
    ones_16 = jnp.ones((CH, 16), jnp.float32)

    deg2 = deg_count(degidx, zeros_16, ones_16).reshape(NC, N, 16)
    h, rp = _tc_in(x, Win, b_in, deg2, ln_g[0], ln_b[0])
    out = None
    for l in range(L):
        agg2 = seg_sum(rp.reshape(NC * N, HD), sidx, didx,
                       zeros_hd).reshape(NC, N, HD)
        if l < L - 1:
            h, rp = _tc_mid(h, agg2, deg2, W1[l], b1[l], W2[l], b2[l],
                            ln_g[l + 1], ln_b[l + 1])
        else:
            out = _tc_out(h, agg2, deg2, W1[l], b1[l], W2[l], b2[l],
                          out_g, out_b, Wout, b_out)
    return out


# speedup vs baseline: 8.1184x; 1.0092x over previous
import functools

import jax
import jax.numpy as jnp
from jax import lax
from jax.experimental import pallas as pl
from jax.experimental.pallas import tpu as pltpu
from jax.experimental.pallas import tpu_sc as plsc

N, E, D, H, L = 10000, 160000, 256, 512, 3
NC, NS = 2, 16
CH = 128
TCHUNKS = 1280
EPAD = TCHUNKS * CH
GARBAGE = N
ACC_ROWS = N + 16
ZPT = ACC_ROWS // NS
RPT = N // NS
HD = D // 2
BN = 1000
GRID = N // BN


@functools.cache
def _sc_kernels():
    mesh = plsc.VectorSubcoreMesh(
        core_axis_name="c", subcore_axis_name="s", num_cores=NC, num_subcores=NS
    )

    params = pltpu.CompilerParams(use_tc_tiling_on_sc=False)

    cpt = TCHUNKS // NS
    nbuf = 4

    @functools.partial(
        pl.kernel,
        out_type=jax.ShapeDtypeStruct((NC * N, HD), jnp.bfloat16),
        mesh=mesh,
        compiler_params=params,
        scratch_types=[
            pltpu.VMEM_SHARED((ACC_ROWS, HD), jnp.bfloat16),
            pltpu.VMEM((cpt, CH), jnp.int32),
            pltpu.VMEM((cpt, CH), jnp.int32),
            [pltpu.VMEM((CH, HD), jnp.bfloat16)] * nbuf,
            [pltpu.SemaphoreType.DMA] * nbuf,
            [pltpu.SemaphoreType.DMA] * nbuf,
        ],
    )
    def _seg_sum(rp_hbm, sidx_hbm, didx_hbm, zeros_hbm, out_hbm,
                 acc, sidx_buf, didx_buf, rows, gsems, ssems):
        c = lax.axis_index("c")
        s = lax.axis_index("s")

        def gather(j, b):
            pltpu.async_copy(rp_hbm.at[sidx_buf.at[j]], rows[b], gsems[b])

        def gather_wait(b):
            pltpu.make_async_copy(rp_hbm.at[sidx_buf.at[0]], rows[b],
                                  gsems[b]).wait()

        def scatter(j, b):
            pltpu.async_copy(rows[b], acc.at[didx_buf.at[j]], ssems[b],
                             add=True)

        def scatter_wait(b):
            pltpu.make_async_copy(rows[b], acc.at[didx_buf.at[0]],
                                  ssems[b]).wait()

        pltpu.sync_copy(zeros_hbm.at[pl.ds(s * ZPT, ZPT)],
                        acc.at[pl.ds(s * ZPT, ZPT)])
        pltpu.sync_copy(sidx_hbm.at[pl.ds(c * TCHUNKS + s * cpt, cpt)],
                        sidx_buf)
        pltpu.sync_copy(didx_hbm.at[pl.ds(s * cpt, cpt)], didx_buf)
        plsc.subcore_barrier()
        for b in range(nbuf - 1):
            gather(b, b)

        @pl.loop(0, cpt, step=nbuf)
        def _group(g):
            for b in range(nbuf):
                j = g + b
                pb = (b + nbuf - 1) % nbuf
                @pl.when(j + nbuf - 1 < cpt)
                def _():
                    @pl.when(j > 0)
                    def _():
                        scatter_wait(pb)

                    gather(j + nbuf - 1, pb)

                gather_wait(b)
                scatter(j, b)

        for b in range(nbuf):
            scatter_wait(b)
        plsc.subcore_barrier()
        pltpu.sync_copy(acc.at[pl.ds(s * RPT, RPT)],
                        out_hbm.at[pl.ds(c * N + s * RPT, RPT)])

    @functools.partial(
        pl.kernel,
        out_type=jax.ShapeDtypeStruct((NC * N, 16), jnp.float32),
        mesh=mesh,
        compiler_params=params,
        scratch_types=[
            pltpu.VMEM_SHARED((ACC_ROWS, 16), jnp.float32),
            pltpu.VMEM((CH,), jnp.int32),
            pltpu.VMEM((CH, 16), jnp.float32),
        ],
    )
    def _deg_count(didx_hbm, zeros_hbm, ones_hbm, out_hbm, acc, idx_v, ones_v):
        c = lax.axis_index("c")
        s = lax.axis_index("s")
        pltpu.sync_copy(zeros_hbm.at[pl.ds(s * ZPT, ZPT)],
                        acc.at[pl.ds(s * ZPT, ZPT)])
        pltpu.sync_copy(ones_hbm, ones_v)
        plsc.subcore_barrier()
        cpt = TCHUNKS // (NC * NS)

        def body(j, carry):
            t = (c * NS + s) * cpt + j
            pltpu.sync_copy(didx_hbm.at[t], idx_v)
            pltpu.sync_copy(ones_v, acc.at[idx_v], add=True)
            return carry

        lax.fori_loop(0, cpt, body, 0)
        plsc.subcore_barrier()
        pltpu.sync_copy(acc.at[pl.ds(s * RPT, RPT)],
                        out_hbm.at[pl.ds(c * N + s * RPT, RPT)])

    return _seg_sum, _deg_count


_SQRT_HALF = 0.7071067811865476


def _gelu(x):
    return 0.5 * x * (1.0 + lax.erf(x * _SQRT_HALF))


def _ln(x, g, b):
    mu = jnp.mean(x, axis=-1, keepdims=True)
    var = jnp.mean((x - mu) ** 2, axis=-1, keepdims=True)
    return (x - mu) * lax.rsqrt(var + 1e-5) * g + b


def _dotT(a, w):
    return lax.dot_general(a, w, (((1,), (1,)), ((), ())),
                           preferred_element_type=jnp.float32)


def _dinv_from(deg_ref):
    deg = deg_ref[0, :, 0:1] + deg_ref[1, :, 0:1]
    return lax.rsqrt(jnp.maximum(deg, 1.0))


def _tc_in_body(x_ref, win_ref, bin_ref, deg_ref, g_ref, b_ref, h_ref, rp_ref):
    h = _gelu(_dotT(x_ref[...], win_ref[...]) + bin_ref[...])
    h_ref[...] = h
    r = (_ln(h, g_ref[...], b_ref[...]) * _dinv_from(deg_ref)
         ).astype(jnp.bfloat16)
    rp_ref[0] = r[:, :HD]
    rp_ref[1] = r[:, HD:]


def _tc_mid_body(h_ref, agg_ref, deg_ref, w1_ref, b1_ref, w2_ref, b2_ref,
                 g_ref, b_ref, hn_ref, rp_ref):
    dinv = _dinv_from(deg_ref)
    agg = jnp.concatenate([agg_ref[0], agg_ref[1]],
                          axis=-1).astype(jnp.float32) * dinv
    f = _gelu(_dotT(agg, w1_ref[...]) + b1_ref[...])
    hn = h_ref[...] + _dotT(f, w2_ref[...]) + b2_ref[...]
    hn_ref[...] = hn
    r = (_ln(hn, g_ref[...], b_ref[...]) * dinv).astype(jnp.bfloat16)
    rp_ref[0] = r[:, :HD]
    rp_ref[1] = r[:, HD:]


def _tc_out_body(h_ref, agg_ref, deg_ref, w1_ref, b1_ref, w2_ref, b2_ref,
                 g_ref, b_ref, wout_ref, bout_ref, o_ref):
    dinv = _dinv_from(deg_ref)
    agg = jnp.concatenate([agg_ref[0], agg_ref[1]],
                          axis=-1).astype(jnp.float32) * dinv
    f = _gelu(_dotT(agg, w1_ref[...]) + b1_ref[...])
    hn = h_ref[...] + _dotT(f, w2_ref[...]) + b2_ref[...]
    r = _ln(hn, g_ref[...], b_ref[...])
    o_ref[...] = _dotT(r, wout_ref[...]) + bout_ref[...]


def _vec_spec(n):
    return pl.BlockSpec((n,), lambda i: (0,))


_ROW_SPEC = pl.BlockSpec((BN, D), lambda i: (i, 0))
_AGG_SPEC = pl.BlockSpec((2, BN, HD), lambda i: (0, i, 0))
_DEG_SPEC = pl.BlockSpec((2, BN, 16), lambda i: (0, i, 0))

_tc_in = pl.pallas_call(
    _tc_in_body,
    grid=(GRID,),
    in_specs=[
        _ROW_SPEC,
        pl.BlockSpec((D, D), lambda i: (0, 0)),
        _vec_spec(D),
        _DEG_SPEC,
        _vec_spec(D),
        _vec_spec(D),
    ],
    out_specs=[_ROW_SPEC, _AGG_SPEC],
    out_shape=[
        jax.ShapeDtypeStruct((N, D), jnp.float32),
        jax.ShapeDtypeStruct((2, N, HD), jnp.bfloat16),
    ],
)

_tc_mid = pl.pallas_call(
    _tc_mid_body,
    grid=(GRID,),
    in_specs=[
        _ROW_SPEC,
        _AGG_SPEC,
        _DEG_SPEC,
        pl.BlockSpec((H, D), lambda i: (0, 0)),
        _vec_spec(H),
        pl.BlockSpec((D, H), lambda i: (0, 0)),
        _vec_spec(D),
        _vec_spec(D),
        _vec_spec(D),
    ],
    out_specs=[_ROW_SPEC, _AGG_SPEC],
    out_shape=[
        jax.ShapeDtypeStruct((N, D), jnp.float32),
        jax.ShapeDtypeStruct((2, N, HD), jnp.bfloat16),
    ],
)

_tc_out = pl.pallas_call(
    _tc_out_body,
    grid=(GRID,),
    in_specs=[
        _ROW_SPEC,
        _AGG_SPEC,
        _DEG_SPEC,
        pl.BlockSpec((H, D), lambda i: (0, 0)),
        _vec_spec(H),
        pl.BlockSpec((D, H), lambda i: (0, 0)),
        _vec_spec(D),
        _vec_spec(D),
        _vec_spec(D),
        pl.BlockSpec((D, D), lambda i: (0, 0)),
        _vec_spec(D),
    ],
    out_specs=_ROW_SPEC,
    out_shape=jax.ShapeDtypeStruct((N, D), jnp.float32),
)


def kernel(x, edge_index, Win, b_in, ln_g, ln_b, W1, b1, W2, b2,
           out_g, out_b, Wout, b_out):
    seg_sum, deg_count = _sc_kernels()
    src = edge_index[0]
    dst = edge_index[1]
    pad = EPAD - E
    src0 = jnp.concatenate([src, jnp.zeros((pad,), jnp.int32)])
    sidx = jnp.concatenate([src0, src0 + N]).reshape(NC * TCHUNKS, CH)
    didx = jnp.concatenate(
        [dst, jnp.full((pad,), GARBAGE, jnp.int32)]).reshape(TCHUNKS, CH)
    degidx = jnp.concatenate(
        [src, jnp.full((pad,), GARBAGE, jnp.int32)]).reshape(TCHUNKS, CH)
    zeros_hd = jnp.zeros((ACC_ROWS, HD), jnp.bfloat16)
    zeros_16 = jnp.zeros((ACC_ROWS, 16), jnp.float32)
    ones_16 = jnp.ones((CH, 16), jnp.float32)

    deg2 = deg_count(degidx, zeros_16, ones_16).reshape(NC, N, 16)
    h, rp = _tc_in(x, Win, b_in, deg2, ln_g[0], ln_b[0])
    out = None
    for l in range(L):
        agg2 = seg_sum(rp.reshape(NC * N, HD), sidx, didx,
                       zeros_hd).reshape(NC, N, HD)
        if l < L - 1:
            h, rp = _tc_mid(h, agg2, deg2, W1[l], b1[l], W2[l], b2[l],
                            ln_g[l + 1], ln_b[l + 1])
        else:
            out = _tc_out(h, agg2, deg2, W1[l], b1[l], W2[l], b2[l],
                          out_g, out_b, Wout, b_out)
    return out


# speedup vs baseline: 8.5041x; 1.0475x over previous
import functools

import jax
import jax.numpy as jnp
from jax import lax
from jax.experimental import pallas as pl
from jax.experimental.pallas import tpu as pltpu
from jax.experimental.pallas import tpu_sc as plsc

N, E, D, H, L = 10000, 160000, 256, 512, 3
NC, NS = 2, 16
CH = 128
TCHUNKS = 1280
EPAD = TCHUNKS * CH
GARBAGE = N
ACC_ROWS = N + 16
ZPT = ACC_ROWS // NS
RPT = N // NS
HD = D // 2
BN = 1000
GRID = N // BN


@functools.cache
def _sc_kernels():
    mesh = plsc.VectorSubcoreMesh(
        core_axis_name="c", subcore_axis_name="s", num_cores=NC, num_subcores=NS
    )

    params = pltpu.CompilerParams(use_tc_tiling_on_sc=False)

    cpt = TCHUNKS // NS
    nbuf = 4

    @functools.partial(
        pl.kernel,
        out_type=(jax.ShapeDtypeStruct((N, HD), jnp.bfloat16),
                  jax.ShapeDtypeStruct((N, HD), jnp.bfloat16)),
        mesh=mesh,
        compiler_params=params,
        scratch_types=[
            pltpu.VMEM_SHARED((ACC_ROWS, HD), jnp.bfloat16),
            pltpu.VMEM((cpt, CH), jnp.int32),
            pltpu.VMEM((cpt, CH), jnp.int32),
            [pltpu.VMEM((CH, HD), jnp.bfloat16)] * nbuf,
            [pltpu.SemaphoreType.DMA] * nbuf,
            [pltpu.SemaphoreType.DMA] * nbuf,
        ],
    )
    def _seg_sum(rp_lo, rp_hi, sidx_hbm, didx_hbm, zeros_hbm,
                 out_lo, out_hi, acc, sidx_buf, didx_buf, rows, gsems, ssems):
        c = lax.axis_index("c")
        s = lax.axis_index("s")

        pltpu.sync_copy(zeros_hbm.at[pl.ds(s * ZPT, ZPT)],
                        acc.at[pl.ds(s * ZPT, ZPT)])
        pltpu.sync_copy(sidx_hbm.at[pl.ds(s * cpt, cpt)], sidx_buf)
        pltpu.sync_copy(didx_hbm.at[pl.ds(s * cpt, cpt)], didx_buf)
        plsc.subcore_barrier()

        def run(rp_hbm, out_hbm):
            def gather(j, b):
                pltpu.async_copy(rp_hbm.at[sidx_buf.at[j]], rows[b], gsems[b])

            def gather_wait(b):
                pltpu.make_async_copy(rp_hbm.at[sidx_buf.at[0]], rows[b],
                                      gsems[b]).wait()

            def scatter(j, b):
                pltpu.async_copy(rows[b], acc.at[didx_buf.at[j]], ssems[b],
                                 add=True)

            def scatter_wait(b):
                pltpu.make_async_copy(rows[b], acc.at[didx_buf.at[0]],
                                      ssems[b]).wait()

            for b in range(nbuf - 1):
                gather(b, b)

            @pl.loop(0, cpt, step=nbuf)
            def _group(g):
                for b in range(nbuf):
                    j = g + b
                    pb = (b + nbuf - 1) % nbuf
                    @pl.when(j + nbuf - 1 < cpt)
                    def _():
                        @pl.when(j > 0)
                        def _():
                            scatter_wait(pb)

                        gather(j + nbuf - 1, pb)

                    gather_wait(b)
                    scatter(j, b)

            for b in range(nbuf):
                scatter_wait(b)
            plsc.subcore_barrier()
            pltpu.sync_copy(acc.at[pl.ds(s * RPT, RPT)],
                            out_hbm.at[pl.ds(s * RPT, RPT)])

        @pl.when(c == 0)
        def _():
            run(rp_lo, out_lo)

        @pl.when(c == 1)
        def _():
            run(rp_hi, out_hi)

    @functools.partial(
        pl.kernel,
        out_type=(jax.ShapeDtypeStruct((N, 16), jnp.float32),
                  jax.ShapeDtypeStruct((N, 16), jnp.float32)),
        mesh=mesh,
        compiler_params=params,
        scratch_types=[
            pltpu.VMEM_SHARED((ACC_ROWS, 16), jnp.float32),
            pltpu.VMEM((CH,), jnp.int32),
            pltpu.VMEM((CH, 16), jnp.float32),
        ],
    )
    def _deg_count(didx_hbm, zeros_hbm, ones_hbm, out_lo, out_hi,
                   acc, idx_v, ones_v):
        c = lax.axis_index("c")
        s = lax.axis_index("s")
        pltpu.sync_copy(zeros_hbm.at[pl.ds(s * ZPT, ZPT)],
                        acc.at[pl.ds(s * ZPT, ZPT)])
        pltpu.sync_copy(ones_hbm, ones_v)
        plsc.subcore_barrier()
        dpt = TCHUNKS // (NC * NS)

        def body(j, carry):
            t = (c * NS + s) * dpt + j
            pltpu.sync_copy(didx_hbm.at[t], idx_v)
            pltpu.sync_copy(ones_v, acc.at[idx_v], add=True)
            return carry

        lax.fori_loop(0, dpt, body, 0)
        plsc.subcore_barrier()

        @pl.when(c == 0)
        def _():
            pltpu.sync_copy(acc.at[pl.ds(s * RPT, RPT)],
                            out_lo.at[pl.ds(s * RPT, RPT)])

        @pl.when(c == 1)
        def _():
            pltpu.sync_copy(acc.at[pl.ds(s * RPT, RPT)],
                            out_hi.at[pl.ds(s * RPT, RPT)])

    return _seg_sum, _deg_count


_SQRT_HALF = 0.7071067811865476


def _gelu(x):
    return 0.5 * x * (1.0 + lax.erf(x * _SQRT_HALF))


def _ln(x, g, b):
    mu = jnp.mean(x, axis=-1, keepdims=True)
    var = jnp.mean((x - mu) ** 2, axis=-1, keepdims=True)
    return (x - mu) * lax.rsqrt(var + 1e-5) * g + b


def _dotT(a, w):
    return lax.dot_general(a, w, (((1,), (1,)), ((), ())),
                           preferred_element_type=jnp.float32)


def _dinv_from(dlo_ref, dhi_ref):
    deg = dlo_ref[:, 0:1] + dhi_ref[:, 0:1]
    return lax.rsqrt(jnp.maximum(deg, 1.0))


def _tc_in_body(x_ref, win_ref, bin_ref, dlo_ref, dhi_ref, g_ref, b_ref,
                h_ref, rlo_ref, rhi_ref):
    h = _gelu(_dotT(x_ref[...], win_ref[...]) + bin_ref[...])
    h_ref[...] = h
    r = (_ln(h, g_ref[...], b_ref[...]) * _dinv_from(dlo_ref, dhi_ref)
         ).astype(jnp.bfloat16)
    rlo_ref[...] = r[:, :HD]
    rhi_ref[...] = r[:, HD:]


def _tc_mid_body(h_ref, alo_ref, ahi_ref, dlo_ref, dhi_ref,
                 w1_ref, b1_ref, w2_ref, b2_ref, g_ref, b_ref,
                 hn_ref, rlo_ref, rhi_ref):
    dinv = _dinv_from(dlo_ref, dhi_ref)
    agg = jnp.concatenate([alo_ref[...], ahi_ref[...]],
                          axis=-1).astype(jnp.float32) * dinv
    f = _gelu(_dotT(agg, w1_ref[...]) + b1_ref[...])
    hn = h_ref[...] + _dotT(f, w2_ref[...]) + b2_ref[...]
    hn_ref[...] = hn
    r = (_ln(hn, g_ref[...], b_ref[...]) * dinv).astype(jnp.bfloat16)
    rlo_ref[...] = r[:, :HD]
    rhi_ref[...] = r[:, HD:]


def _tc_out_body(h_ref, alo_ref, ahi_ref, dlo_ref, dhi_ref,
                 w1_ref, b1_ref, w2_ref, b2_ref, g_ref, b_ref,
                 wout_ref, bout_ref, o_ref):
    dinv = _dinv_from(dlo_ref, dhi_ref)
    agg = jnp.concatenate([alo_ref[...], ahi_ref[...]],
                          axis=-1).astype(jnp.float32) * dinv
    f = _gelu(_dotT(agg, w1_ref[...]) + b1_ref[...])
    hn = h_ref[...] + _dotT(f, w2_ref[...]) + b2_ref[...]
    r = _ln(hn, g_ref[...], b_ref[...])
    o_ref[...] = _dotT(r, wout_ref[...]) + bout_ref[...]


def _vec_spec(n):
    return pl.BlockSpec((n,), lambda i: (0,))


_ROW_SPEC = pl.BlockSpec((BN, D), lambda i: (i, 0))
_HALF_SPEC = pl.BlockSpec((BN, HD), lambda i: (i, 0))
_D16_SPEC = pl.BlockSpec((BN, 16), lambda i: (i, 0))
_RP_SHAPE = jax.ShapeDtypeStruct((N, HD), jnp.bfloat16)

_tc_in = pl.pallas_call(
    _tc_in_body,
    grid=(GRID,),
    in_specs=[
        _ROW_SPEC,
        pl.BlockSpec((D, D), lambda i: (0, 0)),
        _vec_spec(D),
        _D16_SPEC,
        _D16_SPEC,
        _vec_spec(D),
        _vec_spec(D),
    ],
    out_specs=[_ROW_SPEC, _HALF_SPEC, _HALF_SPEC],
    out_shape=[
        jax.ShapeDtypeStruct((N, D), jnp.float32),
        _RP_SHAPE,
        _RP_SHAPE,
    ],
)

_tc_mid = pl.pallas_call(
    _tc_mid_body,
    grid=(GRID,),
    in_specs=[
        _ROW_SPEC,
        _HALF_SPEC,
        _HALF_SPEC,
        _D16_SPEC,
        _D16_SPEC,
        pl.BlockSpec((H, D), lambda i: (0, 0)),
        _vec_spec(H),
        pl.BlockSpec((D, H), lambda i: (0, 0)),
        _vec_spec(D),
        _vec_spec(D),
        _vec_spec(D),
    ],
    out_specs=[_ROW_SPEC, _HALF_SPEC, _HALF_SPEC],
    out_shape=[
        jax.ShapeDtypeStruct((N, D), jnp.float32),
        _RP_SHAPE,
        _RP_SHAPE,
    ],
)

_tc_out = pl.pallas_call(
    _tc_out_body,
    grid=(GRID,),
    in_specs=[
        _ROW_SPEC,
        _HALF_SPEC,
        _HALF_SPEC,
        _D16_SPEC,
        _D16_SPEC,
        pl.BlockSpec((H, D), lambda i: (0, 0)),
        _vec_spec(H),
        pl.BlockSpec((D, H), lambda i: (0, 0)),
        _vec_spec(D),
        _vec_spec(D),
        _vec_spec(D),
        pl.BlockSpec((D, D), lambda i: (0, 0)),
        _vec_spec(D),
    ],
    out_specs=_ROW_SPEC,
    out_shape=jax.ShapeDtypeStruct((N, D), jnp.float32),
)


def kernel(x, edge_index, Win, b_in, ln_g, ln_b, W1, b1, W2, b2,
           out_g, out_b, Wout, b_out):
    seg_sum, deg_count = _sc_kernels()
    src = edge_index[0]
    dst = edge_index[1]
    pad = EPAD - E
    sidx = jnp.concatenate(
        [src, jnp.zeros((pad,), jnp.int32)]).reshape(TCHUNKS, CH)
    didx = jnp.concatenate(
        [dst, jnp.full((pad,), GARBAGE, jnp.int32)]).reshape(TCHUNKS, CH)
    degidx = jnp.concatenate(
        [src, jnp.full((pad,), GARBAGE, jnp.int32)]).reshape(TCHUNKS, CH)
    zeros_hd = jnp.zeros((ACC_ROWS, HD), jnp.bfloat16)
    zeros_16 = jnp.zeros((ACC_ROWS, 16), jnp.float32)
    ones_16 = jnp.ones((CH, 16), jnp.float32)

    dlo, dhi = deg_count(degidx, zeros_16, ones_16)
    h, rlo, rhi = _tc_in(x, Win, b_in, dlo, dhi, ln_g[0], ln_b[0])
    out = None
    for l in range(L):
        alo, ahi = seg_sum(rlo, rhi, sidx, didx, zeros_hd)
        if l < L - 1:
            h, rlo, rhi = _tc_mid(h, alo, ahi, dlo, dhi, W1[l], b1[l],
                                  W2[l], b2[l], ln_g[l + 1], ln_b[l + 1])
        else:
            out = _tc_out(h, alo, ahi, dlo, dhi, W1[l], b1[l], W2[l], b2[l],
                          out_g, out_b, Wout, b_out)
    return out


# speedup vs baseline: 8.5263x; 1.0026x over previous
import functools

import jax
import jax.numpy as jnp
from jax import lax
from jax.experimental import pallas as pl
from jax.experimental.pallas import tpu as pltpu
from jax.experimental.pallas import tpu_sc as plsc

N, E, D, H, L = 10000, 160000, 256, 512, 3
NC, NS = 2, 16
CH = 128
TCHUNKS = 1280
EPAD = TCHUNKS * CH
CHS = 256
SCHUNKS = EPAD // CHS
GARBAGE = N
ACC_ROWS = N + 16
ZPT = ACC_ROWS // NS
RPT = N // NS
HD = D // 2
BN = 1000
GRID = N // BN


@functools.cache
def _sc_kernels():
    mesh = plsc.VectorSubcoreMesh(
        core_axis_name="c", subcore_axis_name="s", num_cores=NC, num_subcores=NS
    )

    params = pltpu.CompilerParams(use_tc_tiling_on_sc=False)

    cpt = SCHUNKS // NS
    nbuf = 4

    @functools.partial(
        pl.kernel,
        out_type=(jax.ShapeDtypeStruct((N, HD), jnp.bfloat16),
                  jax.ShapeDtypeStruct((N, HD), jnp.bfloat16)),
        mesh=mesh,
        compiler_params=params,
        scratch_types=[
            pltpu.VMEM_SHARED((ACC_ROWS, HD), jnp.bfloat16),
            pltpu.VMEM((cpt, CHS), jnp.int32),
            pltpu.VMEM((cpt, CHS), jnp.int32),
            [pltpu.VMEM((CHS, HD), jnp.bfloat16)] * nbuf,
            [pltpu.SemaphoreType.DMA] * nbuf,
            [pltpu.SemaphoreType.DMA] * nbuf,
        ],
    )
    def _seg_sum(rp_lo, rp_hi, sidx_hbm, didx_hbm, zeros_hbm,
                 out_lo, out_hi, acc, sidx_buf, didx_buf, rows, gsems, ssems):
        c = lax.axis_index("c")
        s = lax.axis_index("s")

        pltpu.sync_copy(zeros_hbm.at[pl.ds(s * ZPT, ZPT)],
                        acc.at[pl.ds(s * ZPT, ZPT)])
        pltpu.sync_copy(sidx_hbm.at[pl.ds(s * cpt, cpt)], sidx_buf)
        pltpu.sync_copy(didx_hbm.at[pl.ds(s * cpt, cpt)], didx_buf)
        plsc.subcore_barrier()

        def run(rp_hbm, out_hbm):
            def gather(j, b):
                pltpu.async_copy(rp_hbm.at[sidx_buf.at[j]], rows[b], gsems[b])

            def gather_wait(b):
                pltpu.make_async_copy(rp_hbm.at[sidx_buf.at[0]], rows[b],
                                      gsems[b]).wait()

            def scatter(j, b):
                pltpu.async_copy(rows[b], acc.at[didx_buf.at[j]], ssems[b],
                                 add=True)

            def scatter_wait(b):
                pltpu.make_async_copy(rows[b], acc.at[didx_buf.at[0]],
                                      ssems[b]).wait()

            for b in range(nbuf - 1):
                gather(b, b)

            @pl.loop(0, cpt, step=nbuf)
            def _group(g):
                for b in range(nbuf):
                    j = g + b
                    pb = (b + nbuf - 1) % nbuf
                    @pl.when(j + nbuf - 1 < cpt)
                    def _():
                        @pl.when(j > 0)
                        def _():
                            scatter_wait(pb)

                        gather(j + nbuf - 1, pb)

                    gather_wait(b)
                    scatter(j, b)

            for b in range(nbuf):
                scatter_wait(b)
            plsc.subcore_barrier()
            pltpu.sync_copy(acc.at[pl.ds(s * RPT, RPT)],
                            out_hbm.at[pl.ds(s * RPT, RPT)])

        @pl.when(c == 0)
        def _():
            run(rp_lo, out_lo)

        @pl.when(c == 1)
        def _():
            run(rp_hi, out_hi)

    @functools.partial(
        pl.kernel,
        out_type=(jax.ShapeDtypeStruct((N, 16), jnp.float32),
                  jax.ShapeDtypeStruct((N, 16), jnp.float32)),
        mesh=mesh,
        compiler_params=params,
        scratch_types=[
            pltpu.VMEM_SHARED((ACC_ROWS, 16), jnp.float32),
            pltpu.VMEM((CH,), jnp.int32),
            pltpu.VMEM((CH, 16), jnp.float32),
        ],
    )
    def _deg_count(didx_hbm, zeros_hbm, ones_hbm, out_lo, out_hi,
                   acc, idx_v, ones_v):
        c = lax.axis_index("c")
        s = lax.axis_index("s")
        pltpu.sync_copy(zeros_hbm.at[pl.ds(s * ZPT, ZPT)],
                        acc.at[pl.ds(s * ZPT, ZPT)])
        pltpu.sync_copy(ones_hbm, ones_v)
        plsc.subcore_barrier()
        dpt = TCHUNKS // (NC * NS)

        def body(j, carry):
            t = (c * NS + s) * dpt + j
            pltpu.sync_copy(didx_hbm.at[t], idx_v)
            pltpu.sync_copy(ones_v, acc.at[idx_v], add=True)
            return carry

        lax.fori_loop(0, dpt, body, 0)
        plsc.subcore_barrier()

        @pl.when(c == 0)
        def _():
            pltpu.sync_copy(acc.at[pl.ds(s * RPT, RPT)],
                            out_lo.at[pl.ds(s * RPT, RPT)])

        @pl.when(c == 1)
        def _():
            pltpu.sync_copy(acc.at[pl.ds(s * RPT, RPT)],
                            out_hi.at[pl.ds(s * RPT, RPT)])

    return _seg_sum, _deg_count


_SQRT_HALF = 0.7071067811865476


def _gelu(x):
    return 0.5 * x * (1.0 + lax.erf(x * _SQRT_HALF))


def _ln(x, g, b):
    mu = jnp.mean(x, axis=-1, keepdims=True)
    var = jnp.mean((x - mu) ** 2, axis=-1, keepdims=True)
    return (x - mu) * lax.rsqrt(var + 1e-5) * g + b


def _dotT(a, w):
    return lax.dot_general(a, w, (((1,), (1,)), ((), ())),
                           preferred_element_type=jnp.float32)


def _dinv_from(dlo_ref, dhi_ref):
    deg = dlo_ref[:, 0:1] + dhi_ref[:, 0:1]
    return lax.rsqrt(jnp.maximum(deg, 1.0))


def _tc_in_body(x_ref, win_ref, bin_ref, dlo_ref, dhi_ref, g_ref, b_ref,
                h_ref, rlo_ref, rhi_ref):
    h = _gelu(_dotT(x_ref[...], win_ref[...]) + bin_ref[...])
    h_ref[...] = h
    r = (_ln(h, g_ref[...], b_ref[...]) * _dinv_from(dlo_ref, dhi_ref)
         ).astype(jnp.bfloat16)
    rlo_ref[...] = r[:, :HD]
    rhi_ref[...] = r[:, HD:]


def _tc_mid_body(h_ref, alo_ref, ahi_ref, dlo_ref, dhi_ref,
                 w1_ref, b1_ref, w2_ref, b2_ref, g_ref, b_ref,
                 hn_ref, rlo_ref, rhi_ref):
    dinv = _dinv_from(dlo_ref, dhi_ref)
    agg = jnp.concatenate([alo_ref[...], ahi_ref[...]],
                          axis=-1).astype(jnp.float32) * dinv
    f = _gelu(_dotT(agg, w1_ref[...]) + b1_ref[...])
    hn = h_ref[...] + _dotT(f, w2_ref[...]) + b2_ref[...]
    hn_ref[...] = hn
    r = (_ln(hn, g_ref[...], b_ref[...]) * dinv).astype(jnp.bfloat16)
    rlo_ref[...] = r[:, :HD]
    rhi_ref[...] = r[:, HD:]


def _tc_out_body(h_ref, alo_ref, ahi_ref, dlo_ref, dhi_ref,
                 w1_ref, b1_ref, w2_ref, b2_ref, g_ref, b_ref,
                 wout_ref, bout_ref, o_ref):
    dinv = _dinv_from(dlo_ref, dhi_ref)
    agg = jnp.concatenate([alo_ref[...], ahi_ref[...]],
                          axis=-1).astype(jnp.float32) * dinv
    f = _gelu(_dotT(agg, w1_ref[...]) + b1_ref[...])
    hn = h_ref[...] + _dotT(f, w2_ref[...]) + b2_ref[...]
    r = _ln(hn, g_ref[...], b_ref[...])
    o_ref[...] = _dotT(r, wout_ref[...]) + bout_ref[...]


def _vec_spec(n):
    return pl.BlockSpec((n,), lambda i: (0,))


_ROW_SPEC = pl.BlockSpec((BN, D), lambda i: (i, 0))
_HALF_SPEC = pl.BlockSpec((BN, HD), lambda i: (i, 0))
_D16_SPEC = pl.BlockSpec((BN, 16), lambda i: (i, 0))
_RP_SHAPE = jax.ShapeDtypeStruct((N, HD), jnp.bfloat16)

_tc_in = pl.pallas_call(
    _tc_in_body,
    grid=(GRID,),
    in_specs=[
        _ROW_SPEC,
        pl.BlockSpec((D, D), lambda i: (0, 0)),
        _vec_spec(D),
        _D16_SPEC,
        _D16_SPEC,
        _vec_spec(D),
        _vec_spec(D),
    ],
    out_specs=[_ROW_SPEC, _HALF_SPEC, _HALF_SPEC],
    out_shape=[
        jax.ShapeDtypeStruct((N, D), jnp.float32),
        _RP_SHAPE,
        _RP_SHAPE,
    ],
)

_tc_mid = pl.pallas_call(
    _tc_mid_body,
    grid=(GRID,),
    in_specs=[
        _ROW_SPEC,
        _HALF_SPEC,
        _HALF_SPEC,
        _D16_SPEC,
        _D16_SPEC,
        pl.BlockSpec((H, D), lambda i: (0, 0)),
        _vec_spec(H),
        pl.BlockSpec((D, H), lambda i: (0, 0)),
        _vec_spec(D),
        _vec_spec(D),
        _vec_spec(D),
    ],
    out_specs=[_ROW_SPEC, _HALF_SPEC, _HALF_SPEC],
    out_shape=[
        jax.ShapeDtypeStruct((N, D), jnp.float32),
        _RP_SHAPE,
        _RP_SHAPE,
    ],
)

_tc_out = pl.pallas_call(
    _tc_out_body,
    grid=(GRID,),
    in_specs=[
        _ROW_SPEC,
        _HALF_SPEC,
        _HALF_SPEC,
        _D16_SPEC,
        _D16_SPEC,
        pl.BlockSpec((H, D), lambda i: (0, 0)),
        _vec_spec(H),
        pl.BlockSpec((D, H), lambda i: (0, 0)),
        _vec_spec(D),
        _vec_spec(D),
        _vec_spec(D),
        pl.BlockSpec((D, D), lambda i: (0, 0)),
        _vec_spec(D),
    ],
    out_specs=_ROW_SPEC,
    out_shape=jax.ShapeDtypeStruct((N, D), jnp.float32),
)


def kernel(x, edge_index, Win, b_in, ln_g, ln_b, W1, b1, W2, b2,
           out_g, out_b, Wout, b_out):
    seg_sum, deg_count = _sc_kernels()
    src = edge_index[0]
    dst = edge_index[1]
    pad = EPAD - E
    sidx = jnp.concatenate(
        [src, jnp.zeros((pad,), jnp.int32)]).reshape(SCHUNKS, CHS)
    didx = jnp.concatenate(
        [dst, jnp.full((pad,), GARBAGE, jnp.int32)]).reshape(SCHUNKS, CHS)
    degidx = jnp.concatenate(
        [src, jnp.full((pad,), GARBAGE, jnp.int32)]).reshape(TCHUNKS, CH)
    zeros_hd = jnp.zeros((ACC_ROWS, HD), jnp.bfloat16)
    zeros_16 = jnp.zeros((ACC_ROWS, 16), jnp.float32)
    ones_16 = jnp.ones((CH, 16), jnp.float32)

    dlo, dhi = deg_count(degidx, zeros_16, ones_16)
    h, rlo, rhi = _tc_in(x, Win, b_in, dlo, dhi, ln_g[0], ln_b[0])
    out = None
    for l in range(L):
        alo, ahi = seg_sum(rlo, rhi, sidx, didx, zeros_hd)
        if l < L - 1:
            h, rlo, rhi = _tc_mid(h, alo, ahi, dlo, dhi, W1[l], b1[l],
                                  W2[l], b2[l], ln_g[l + 1], ln_b[l + 1])
        else:
            out = _tc_out(h, alo, ahi, dlo, dhi, W1[l], b1[l], W2[l], b2[l],
                          out_g, out_b, Wout, b_out)
    return out
